# baseline (speedup 1.0000x reference)
import jax
import jax.numpy as jnp
from jax.experimental import pallas as pl
from jax.experimental.pallas import tpu as pltpu

IN_W = 2048
OUT_W = 2048
P = 8
FANOUT = 4
G = 85
N_HEAD_G = 21
GPAD = 32
SEG = P * GPAD
HEAD_W = FANOUT * SEG
REST_W = P * (G - N_HEAD_G) * FANOUT
TOT_W = HEAD_W + REST_W

TB = 512


def _gelu_exact(x):
    return 0.5 * x * (1.0 + jax.lax.erf(x * (2.0 ** -0.5)))


def _ffff_body(x_ref, w1_ref, b1_ref, w2_ref, sseg_ref, sexp_ref, o_ref):
    x_bf = x_ref[...]

    z = jax.lax.dot_general(
        x_bf, w1_ref[...], (((1,), (1,)), ((), ())),
        preferred_element_type=jnp.float32)
    a = _gelu_exact(z + b1_ref[...])
    a_rest = a[:, HEAD_W:]

    a0 = a[:, 0 * SEG:1 * SEG]
    a1 = a[:, 1 * SEG:2 * SEG]
    a2 = a[:, 2 * SEG:3 * SEG]
    a3 = a[:, 3 * SEG:4 * SEG]
    one = jnp.float32(1.0)
    dec = jnp.where(a1 > a0, one, 0.0)
    m = jnp.maximum(a0, a1)
    dec = jnp.where(a2 > m, 2.0, dec)
    m = jnp.maximum(m, a2)
    dec = jnp.where(a3 > m, 3.0, dec)

    sseg = sseg_ref[...]
    sexp = sexp_ref[...]

    def segb(v):
        return jax.lax.dot_general(
            v.astype(jnp.bfloat16), sseg, (((1,), (0,)), ((), ())),
            preferred_element_type=jnp.float32)

    gio = jax.lax.rem(
        jax.lax.broadcasted_iota(jnp.int32, (1, SEG), 1), GPAD
    ).astype(jnp.float32)

    oh0 = jnp.where(gio == 0.0, one, 0.0)
    g1 = 1.0 + segb(dec * oh0)
    oh1 = jnp.where(gio == g1, one, 0.0)
    g2 = 1.0 + 4.0 * g1 + segb(dec * oh1)
    oh2 = jnp.where(gio == g2, one, 0.0)
    g3 = 1.0 + 4.0 * g2 + segb(dec * oh2)

    hmask = oh0 + oh1 + oh2
    hm = [(af * hmask).astype(jnp.bfloat16) for af in (a0, a1, a2, a3)]

    g3e = jax.lax.dot_general(
        g3.astype(jnp.bfloat16), sexp, (((1,), (0,)), ((), ())),
        preferred_element_type=jnp.float32)
    rio = jax.lax.broadcasted_iota(jnp.int32, (1, REST_W), 1)
    r_g = (21 + jax.lax.div(jax.lax.rem(rio, 256), FANOUT)).astype(
        jnp.float32)
    am_rest = jnp.where(r_g == g3e, a_rest, 0.0).astype(jnp.bfloat16)

    am = jnp.concatenate(hm + [am_rest], axis=1)
    o_ref[...] = jax.lax.dot_general(
        am, w2_ref[...], (((1,), (1,)), ((), ())),
        preferred_element_type=jnp.float32)


@jax.jit
def _ffff(x, W1, b1, W2, Sseg, Sexp):
    B = x.shape[0]
    grid = (B // TB,)
    return pl.pallas_call(
        _ffff_body,
        grid=grid,
        in_specs=[
            pl.BlockSpec((TB, IN_W), lambda i: (i, 0)),
            pl.BlockSpec((TOT_W, IN_W), lambda i: (0, 0)),
            pl.BlockSpec((1, TOT_W), lambda i: (0, 0)),
            pl.BlockSpec((OUT_W, TOT_W), lambda i: (0, 0)),
            pl.BlockSpec((SEG, SEG), lambda i: (0, 0)),
            pl.BlockSpec((SEG, REST_W), lambda i: (0, 0)),
        ],
        out_specs=pl.BlockSpec((TB, OUT_W), lambda i: (i, 0)),
        out_shape=jax.ShapeDtypeStruct((B, OUT_W), jnp.float32),
        compiler_params=pltpu.CompilerParams(
            dimension_semantics=("parallel",)),
    )(x, W1, b1, W2, Sseg, Sexp)


def kernel(oldx, W_in, b_in, W_out):
    x = oldx.reshape(-1, IN_W).astype(jnp.bfloat16)

    Wi = W_in.reshape(P, G, FANOUT, IN_W)
    bi = b_in.reshape(P, G, FANOUT)
    Wh = jnp.transpose(Wi[:, :N_HEAD_G], (2, 0, 1, 3))
    Wh = jnp.pad(Wh, ((0, 0), (0, 0), (0, GPAD - N_HEAD_G), (0, 0)))
    bh = jnp.transpose(bi[:, :N_HEAD_G], (2, 0, 1))
    bh = jnp.pad(bh, ((0, 0), (0, 0), (0, GPAD - N_HEAD_G)))
    W1 = jnp.concatenate(
        [Wh.reshape(HEAD_W, IN_W),
         Wi[:, N_HEAD_G:].reshape(REST_W, IN_W)]).astype(jnp.bfloat16)
    b1 = jnp.concatenate(
        [bh.reshape(HEAD_W), bi[:, N_HEAD_G:].reshape(REST_W)]
    ).reshape(1, TOT_W)

    Wo = W_out.reshape(OUT_W, P, G, FANOUT)
    Woh = jnp.transpose(Wo[:, :, :N_HEAD_G], (0, 3, 1, 2))
    Woh = jnp.pad(Woh, ((0, 0), (0, 0), (0, 0), (0, GPAD - N_HEAD_G)))
    W2 = jnp.concatenate(
        [Woh.reshape(OUT_W, HEAD_W),
         Wo[:, :, N_HEAD_G:].reshape(OUT_W, REST_W)],
        axis=1).astype(jnp.bfloat16)

    iseg = jnp.arange(SEG)
    Sseg = (iseg[:, None] // GPAD == iseg[None, :] // GPAD).astype(
        jnp.bfloat16)
    irest = jnp.arange(REST_W)
    Sexp = ((iseg[:, None] == (irest[None, :] // 256) * GPAD)).astype(
        jnp.bfloat16)

    out = _ffff(x, W1, b1, W2, Sseg, Sexp)
    return out.reshape(oldx.shape)


# speedup vs baseline: 1.1586x; 1.1586x over previous
import jax
import jax.numpy as jnp
from jax.experimental import pallas as pl
from jax.experimental.pallas import tpu as pltpu

IN_W = 2048
OUT_W = 2048
P = 8
FANOUT = 4
G = 85
NPG = G * FANOUT
N_HEAD_G = 21
GPAD = 32
SEG = P * GPAD
HEAD_W = FANOUT * SEG
TOT_W = P * NPG

TB = 512


def _gelu_exact(x):
    return 0.5 * x * (1.0 + jax.lax.erf(x * (2.0 ** -0.5)))


def _ffff_body(x_ref, w1_ref, w1h_ref, b1_ref, bh_ref, w2_ref, sseg_ref,
               sexp_ref, o_ref):
    x_bf = x_ref[...].astype(jnp.bfloat16)

    z = jax.lax.dot_general(
        x_bf, w1_ref[...], (((1,), (1,)), ((), ())),
        preferred_element_type=jnp.float32)
    a = _gelu_exact(z + b1_ref[...])

    zh = jax.lax.dot_general(
        x_bf, w1h_ref[...], (((1,), (1,)), ((), ())),
        preferred_element_type=jnp.float32)
    ah = _gelu_exact(zh + bh_ref[...])

    a0 = ah[:, 0 * SEG:1 * SEG]
    a1 = ah[:, 1 * SEG:2 * SEG]
    a2 = ah[:, 2 * SEG:3 * SEG]
    a3 = ah[:, 3 * SEG:4 * SEG]
    one = jnp.float32(1.0)
    dec = jnp.where(a1 > a0, one, 0.0)
    m = jnp.maximum(a0, a1)
    dec = jnp.where(a2 > m, 2.0, dec)
    m = jnp.maximum(m, a2)
    dec = jnp.where(a3 > m, 3.0, dec)

    sseg = sseg_ref[...]
    sexp = sexp_ref[...]

    def segb(v):
        return jax.lax.dot_general(
            v.astype(jnp.bfloat16), sseg, (((1,), (0,)), ((), ())),
            preferred_element_type=jnp.float32)

    gio = jax.lax.rem(
        jax.lax.broadcasted_iota(jnp.int32, (1, SEG), 1), GPAD
    ).astype(jnp.float32)

    oh0 = jnp.where(gio == 0.0, one, 0.0)
    g1 = 1.0 + segb(dec * oh0)
    oh1 = jnp.where(gio == g1, one, 0.0)
    g2 = 1.0 + 4.0 * g1 + segb(dec * oh1)
    oh2 = jnp.where(gio == g2, one, 0.0)
    g3 = 1.0 + 4.0 * g2 + segb(dec * oh2)

    g1e = jax.lax.dot_general(
        g1.astype(jnp.bfloat16), sexp, (((1,), (0,)), ((), ())),
        preferred_element_type=jnp.float32)
    g2e = jax.lax.dot_general(
        g2.astype(jnp.bfloat16), sexp, (((1,), (0,)), ((), ())),
        preferred_element_type=jnp.float32)
    g3e = jax.lax.dot_general(
        g3.astype(jnp.bfloat16), sexp, (((1,), (0,)), ((), ())),
        preferred_element_type=jnp.float32)

    jio = jax.lax.broadcasted_iota(jnp.int32, (1, TOT_W), 1)
    gj = jax.lax.div(jax.lax.rem(jio, NPG), FANOUT).astype(jnp.float32)
    keep = ((gj == 0.0) | (gj == g1e) | (gj == g2e) | (gj == g3e))
    am = jnp.where(keep, a, 0.0).astype(jnp.bfloat16)

    o_ref[...] = jax.lax.dot_general(
        am, w2_ref[...], (((1,), (1,)), ((), ())),
        preferred_element_type=jnp.float32)


@jax.jit
def _ffff(x, W1, W1h, b1, bh, W2, Sseg, Sexp):
    B = x.shape[0]
    grid = (B // TB,)
    return pl.pallas_call(
        _ffff_body,
        grid=grid,
        in_specs=[
            pl.BlockSpec((TB, IN_W), lambda i: (i, 0)),
            pl.BlockSpec((TOT_W, IN_W), lambda i: (0, 0)),
            pl.BlockSpec((HEAD_W, IN_W), lambda i: (0, 0)),
            pl.BlockSpec((1, TOT_W), lambda i: (0, 0)),
            pl.BlockSpec((1, HEAD_W), lambda i: (0, 0)),
            pl.BlockSpec((OUT_W, TOT_W), lambda i: (0, 0)),
            pl.BlockSpec((SEG, SEG), lambda i: (0, 0)),
            pl.BlockSpec((SEG, TOT_W), lambda i: (0, 0)),
        ],
        out_specs=pl.BlockSpec((TB, OUT_W), lambda i: (i, 0)),
        out_shape=jax.ShapeDtypeStruct((B, OUT_W), jnp.float32),
        compiler_params=pltpu.CompilerParams(
            dimension_semantics=("parallel",)),
    )(x, W1, W1h, b1, bh, W2, Sseg, Sexp)


def kernel(oldx, W_in, b_in, W_out):
    x = oldx.reshape(-1, IN_W)

    W1 = W_in.astype(jnp.bfloat16)
    W2 = W_out.astype(jnp.bfloat16)
    b1 = b_in.reshape(1, TOT_W)

    Wi = W_in.reshape(P, G, FANOUT, IN_W)
    bi = b_in.reshape(P, G, FANOUT)
    Wh = jnp.transpose(Wi[:, :N_HEAD_G], (2, 0, 1, 3))
    Wh = jnp.pad(Wh, ((0, 0), (0, 0), (0, GPAD - N_HEAD_G), (0, 0)))
    W1h = Wh.reshape(HEAD_W, IN_W).astype(jnp.bfloat16)
    bhh = jnp.transpose(bi[:, :N_HEAD_G], (2, 0, 1))
    bhh = jnp.pad(bhh, ((0, 0), (0, 0), (0, GPAD - N_HEAD_G)))
    bh = bhh.reshape(1, HEAD_W)

    iseg = jnp.arange(SEG)
    Sseg = (iseg[:, None] // GPAD == iseg[None, :] // GPAD).astype(
        jnp.bfloat16)
    itot = jnp.arange(TOT_W)
    Sexp = (iseg[:, None] == (itot[None, :] // NPG) * GPAD).astype(
        jnp.bfloat16)

    out = _ffff(x, W1, W1h, b1, bh, W2, Sseg, Sexp)
    return out.reshape(oldx.shape)


# speedup vs baseline: 1.2583x; 1.0861x over previous
import jax
import jax.numpy as jnp
from jax.experimental import pallas as pl
from jax.experimental.pallas import tpu as pltpu

IN_W = 2048
OUT_W = 2048
P = 8
FANOUT = 4
G = 85
NPG = G * FANOUT
N_HEAD_G = 21
GPAD = 32
SEG = P * GPAD
HEAD_W = FANOUT * SEG
TOT_W = P * NPG

TB = 512


def _gelu_exact(x):
    return 0.5 * x * (1.0 + jax.lax.erf(x * (2.0 ** -0.5)))


def _gelu_tanh(x):
    c = 0.7978845608028654
    return 0.5 * x * (1.0 + jnp.tanh(c * (x + 0.044715 * x * x * x)))


def _ffff_body(x_ref, w1_ref, w1h_ref, b1_ref, bh_ref, w2_ref, sseg_ref,
               sexp_ref, lo_ref, hi_ref, o_ref):
    x_bf = x_ref[...].astype(jnp.bfloat16)

    z = jax.lax.dot_general(
        x_bf, w1_ref[...], (((1,), (1,)), ((), ())),
        preferred_element_type=jnp.float32)
    a = _gelu_tanh(z + b1_ref[...])

    zh = jax.lax.dot_general(
        x_bf, w1h_ref[...], (((1,), (1,)), ((), ())),
        preferred_element_type=jnp.float32)
    ah = _gelu_exact(zh + bh_ref[...])

    a0 = ah[:, 0 * SEG:1 * SEG]
    a1 = ah[:, 1 * SEG:2 * SEG]
    a2 = ah[:, 2 * SEG:3 * SEG]
    a3 = ah[:, 3 * SEG:4 * SEG]
    one = jnp.float32(1.0)
    dec = jnp.where(a1 > a0, one, 0.0)
    m = jnp.maximum(a0, a1)
    dec = jnp.where(a2 > m, 2.0, dec)
    m = jnp.maximum(m, a2)
    dec = jnp.where(a3 > m, 3.0, dec)

    sseg = sseg_ref[...]
    sexp = sexp_ref[...]

    def segb(v):
        return jax.lax.dot_general(
            v.astype(jnp.bfloat16), sseg, (((1,), (0,)), ((), ())),
            preferred_element_type=jnp.float32)

    gio = jax.lax.rem(
        jax.lax.broadcasted_iota(jnp.int32, (1, SEG), 1), GPAD
    ).astype(jnp.float32)

    oh0 = jnp.where(gio == 0.0, one, 0.0)
    g1 = 1.0 + segb(dec * oh0)
    oh1 = jnp.where(gio == g1, one, 0.0)
    g2 = 1.0 + 4.0 * g1 + segb(dec * oh1)
    oh2 = jnp.where(gio == g2, one, 0.0)
    g3 = 1.0 + 4.0 * g2 + segb(dec * oh2)

    g3e = jax.lax.dot_general(
        g3.astype(jnp.bfloat16), sexp, (((1,), (0,)), ((), ())),
        preferred_element_type=jnp.float32)
    keep = (g3e >= lo_ref[...]) & (g3e < hi_ref[...])
    am = jnp.where(keep, a, 0.0).astype(jnp.bfloat16)

    o_ref[...] = jax.lax.dot_general(
        am, w2_ref[...], (((1,), (1,)), ((), ())),
        preferred_element_type=jnp.float32)


@jax.jit
def _ffff(x, W1, W1h, b1, bh, W2, Sseg, Sexp, lo, hi):
    B = x.shape[0]
    grid = (B // TB,)
    return pl.pallas_call(
        _ffff_body,
        grid=grid,
        in_specs=[
            pl.BlockSpec((TB, IN_W), lambda i: (i, 0)),
            pl.BlockSpec((TOT_W, IN_W), lambda i: (0, 0)),
            pl.BlockSpec((HEAD_W, IN_W), lambda i: (0, 0)),
            pl.BlockSpec((1, TOT_W), lambda i: (0, 0)),
            pl.BlockSpec((1, HEAD_W), lambda i: (0, 0)),
            pl.BlockSpec((OUT_W, TOT_W), lambda i: (0, 0)),
            pl.BlockSpec((SEG, SEG), lambda i: (0, 0)),
            pl.BlockSpec((SEG, TOT_W), lambda i: (0, 0)),
            pl.BlockSpec((1, TOT_W), lambda i: (0, 0)),
            pl.BlockSpec((1, TOT_W), lambda i: (0, 0)),
        ],
        out_specs=pl.BlockSpec((TB, OUT_W), lambda i: (i, 0)),
        out_shape=jax.ShapeDtypeStruct((B, OUT_W), jnp.float32),
        compiler_params=pltpu.CompilerParams(
            dimension_semantics=("parallel",)),
    )(x, W1, W1h, b1, bh, W2, Sseg, Sexp, lo, hi)


def kernel(oldx, W_in, b_in, W_out):
    x = oldx.reshape(-1, IN_W)

    W1 = W_in.astype(jnp.bfloat16)
    W2 = W_out.astype(jnp.bfloat16)
    b1 = b_in.reshape(1, TOT_W)

    Wi = W_in.reshape(P, G, FANOUT, IN_W)
    bi = b_in.reshape(P, G, FANOUT)
    Wh = jnp.transpose(Wi[:, :N_HEAD_G], (2, 0, 1, 3))
    Wh = jnp.pad(Wh, ((0, 0), (0, 0), (0, GPAD - N_HEAD_G), (0, 0)))
    W1h = Wh.reshape(HEAD_W, IN_W).astype(jnp.bfloat16)
    bhh = jnp.transpose(bi[:, :N_HEAD_G], (2, 0, 1))
    bhh = jnp.pad(bhh, ((0, 0), (0, 0), (0, GPAD - N_HEAD_G)))
    bh = bhh.reshape(1, HEAD_W)

    iseg = jnp.arange(SEG)
    Sseg = (iseg[:, None] // GPAD == iseg[None, :] // GPAD).astype(
        jnp.bfloat16)
    itot = jnp.arange(TOT_W)
    Sexp = (iseg[:, None] == (itot[None, :] // NPG) * GPAD).astype(
        jnp.bfloat16)

    gj = (itot % NPG) // FANOUT
    lo = jnp.where(gj == 0, 21,
                   jnp.where(gj < 5, 21 + 16 * (gj - 1),
                             jnp.where(gj < 21, 21 + 4 * (gj - 5), gj)))
    hi = jnp.where(gj == 0, 85,
                   jnp.where(gj < 5, 21 + 16 * gj,
                             jnp.where(gj < 21, 25 + 4 * (gj - 5), gj + 1)))
    lo = lo.astype(jnp.float32).reshape(1, TOT_W)
    hi = hi.astype(jnp.float32).reshape(1, TOT_W)

    out = _ffff(x, W1, W1h, b1, bh, W2, Sseg, Sexp, lo, hi)
    return out.reshape(oldx.shape)
